# baseline (speedup 1.0000x reference)
import functools

import jax
import jax.numpy as jnp
from jax import lax
from jax.experimental import pallas as pl
from jax.experimental.pallas import tpu as pltpu
from jax.experimental.pallas import tpu_sc as plsc

N, E, D = 10000, 320000, 128

NC, NS, L = 2, 16, 16
NW = NC * NS
EPW = E // NW
C = 64
KB = 1
KBC = KB * C
NSS = 160
KCH = NSS * KB
PADE = NSS * KBC - EPW
RPT = 624
RSZ = 640
NSL = D // L
NTRASH = 8


def _sc_edge_body(hp_hbm, pk_hbm, ea0_hbm, ea1_hbm, we_hbm, be_hbm,
                  zeros_hbm, out_hbm,
                  pkbuf, srcidx, dstidx, ea0v, ea1v, wev, bev, rows16, rows,
                  agg_sh, gsem, esem, ssem, psem):
    cid = lax.axis_index("c")
    sid = lax.axis_index("s")
    wid = sid * NC + cid

    pltpu.sync_copy(zeros_hbm.at[pl.ds(sid * RPT, RSZ)],
                    agg_sh.at[pl.ds(sid * RPT, RSZ)])

    pltpu.sync_copy(we_hbm, wev)
    pltpu.sync_copy(be_hbm, bev)
    w0 = [wev[0, pl.ds(L * s, L)] for s in range(NSL)]
    w1 = [wev[1, pl.ds(L * s, L)] for s in range(NSL)]
    bb = [bev[pl.ds(L * s, L)] for s in range(NSL)]

    plsc.subcore_barrier()

    def fire_pk(bset, ss):
        pltpu.async_copy(pk_hbm.at[wid, ss], pkbuf.at[bset], psem.at[bset])

    def drain_pk(bset):
        pltpu.make_async_copy(pk_hbm.at[0, 0], pkbuf.at[bset],
                              psem.at[bset]).wait()

    def unpack(bset, ss):
        del ss
        for c in range(KB):
            for g in range(C // L):
                v = pkbuf[bset, c, pl.ds(g * L, L)]
                srcidx[bset, c, pl.ds(g * L, L)] = jnp.bitwise_and(v, 0xFFFF)
                dstidx[bset, c, pl.ds(g * L, L)] = jnp.right_shift(v, 16)

    def fire(bset, ss):
        pltpu.async_copy(ea0_hbm.at[wid, ss], ea0v.at[bset], esem.at[bset])
        pltpu.async_copy(ea1_hbm.at[wid, ss], ea1v.at[bset], esem.at[bset])
        for c in range(KB):
            pltpu.async_copy(hp_hbm.at[srcidx.at[bset, c]],
                             rows16.at[bset, c], gsem.at[bset])

    def drain_in(bset):
        pltpu.make_async_copy(ea0_hbm.at[0, 0], ea0v.at[bset],
                              esem.at[bset]).wait()
        pltpu.make_async_copy(ea1_hbm.at[0, 0], ea1v.at[bset],
                              esem.at[bset]).wait()
        for c in range(KB):
            pltpu.make_async_copy(hp_hbm.at[srcidx.at[bset, c]],
                                  rows16.at[bset, c], gsem.at[bset]).wait()

    def drain_scatters(bset):
        for c in range(KB):
            pltpu.make_async_copy(rows.at[bset, c],
                                  agg_sh.at[dstidx.at[bset, c]],
                                  ssem.at[bset]).wait()

    def compute(bset, c):
        def group_body(g, carry):
            j0 = g * L
            a0v = ea0v[bset, pl.ds(c * C + j0, L)]
            a1v = ea1v[bset, pl.ds(c * C + j0, L)]
            for jj in range(L):
                a0 = a0v[jj]
                a1 = a1v[jj]
                for s in range(NSL // 2):
                    vp = rows16[bset, c, j0 + jj, pl.ds(L * s, L)]
                    lo, hi = plsc.unpack(plsc.bitcast(vp, jnp.bfloat16),
                                         format=plsc.PackFormat.INTERLEAVED)
                    sh = s + NSL // 2
                    vl = lo + (a0 * w0[s] + (a1 * w1[s] + bb[s]))
                    vh = hi + (a0 * w0[sh] + (a1 * w1[sh] + bb[sh]))
                    rows[bset, c, j0 + jj, pl.ds(L * s, L)] = (
                        jnp.maximum(vl, 0.0))
                    rows[bset, c, j0 + jj, pl.ds(L * sh, L)] = (
                        jnp.maximum(vh, 0.0))
            return carry

        lax.fori_loop(0, C // L, group_body, 0, unroll=False)

    fire_pk(0, 0)
    drain_pk(0)
    unpack(0, 0)
    fire(0, 0)
    fire_pk(1, 1)

    def outer(t, carry):
        for bset in range(2):
            ss = 2 * t + bset
            other = 1 - bset

            @pl.when(ss + 2 < NSS)
            def _():
                fire_pk(bset, ss + 2)

            @pl.when(ss + 1 < NSS)
            def _():
                @pl.when(ss >= 1)
                def _():
                    drain_scatters(other)
                drain_pk(other)
                unpack(other, ss + 1)
                fire(other, ss + 1)

            drain_in(bset)
            for c in range(KB):
                compute(bset, c)
                pltpu.async_copy(rows.at[bset, c],
                                 agg_sh.at[dstidx.at[bset, c]],
                                 ssem.at[bset], add=True)
        return carry

    lax.fori_loop(0, NSS // 2, outer, 0, unroll=False)
    drain_scatters(0)
    drain_scatters(1)

    plsc.subcore_barrier()
    pltpu.sync_copy(agg_sh.at[pl.ds(sid * RPT, RSZ)],
                    out_hbm.at[cid, pl.ds(sid * RPT, RSZ)])


_sc_edge = functools.partial(
    pl.kernel,
    out_type=jax.ShapeDtypeStruct((NC, N, D), jnp.float32),
    compiler_params=pltpu.CompilerParams(needs_layout_passes=False,
                                         use_tc_tiling_on_sc=False),
    mesh=plsc.VectorSubcoreMesh(core_axis_name="c", subcore_axis_name="s",
                                num_cores=NC, num_subcores=NS),
    scratch_types=[
        pltpu.VMEM((2, KB, C), jnp.int32),
        pltpu.VMEM((2, KB, C), jnp.int32),
        pltpu.VMEM((2, KB, C), jnp.int32),
        pltpu.VMEM((2, KBC), jnp.float32),
        pltpu.VMEM((2, KBC), jnp.float32),
        pltpu.VMEM((2, D), jnp.float32),
        pltpu.VMEM((D,), jnp.float32),
        pltpu.VMEM((2, KB, C, D // 2), jnp.uint32),
        pltpu.VMEM((2, KB, C, D), jnp.float32),
        pltpu.VMEM_SHARED((N + NTRASH, D), jnp.float32),
        pltpu.SemaphoreType.DMA((2,)),
        pltpu.SemaphoreType.DMA((2,)),
        pltpu.SemaphoreType.DMA((2,)),
        pltpu.SemaphoreType.DMA((2,)),
    ],
)(_sc_edge_body)


def _pack_halves(y):
    lo = jax.lax.bitcast_convert_type(y[:, :D // 2].astype(jnp.bfloat16),
                                      jnp.uint16).astype(jnp.uint32)
    hi = jax.lax.bitcast_convert_type(y[:, D // 2:].astype(jnp.bfloat16),
                                      jnp.uint16).astype(jnp.uint32)
    return jnp.bitwise_or(lo, jnp.left_shift(hi, 16))


def _tc_node_body(h_ref, p_ref, w_ref, b_ref, o_ref, op_ref):
    t = h_ref[...] + p_ref[0] + p_ref[1]
    y = jnp.dot(t, w_ref[...], preferred_element_type=jnp.float32) + b_ref[...]
    y = jnp.where(y >= 0.0, y, 0.01 * y)
    o_ref[...] = y
    op_ref[...] = _pack_halves(y)


_RB = 1000


def _tc_node(h, parts, w, b):
    return pl.pallas_call(
        _tc_node_body,
        grid=(N // _RB,),
        in_specs=[
            pl.BlockSpec((_RB, D), lambda i: (i, 0)),
            pl.BlockSpec((NC, _RB, D), lambda i: (0, i, 0)),
            pl.BlockSpec((D, D), lambda i: (0, 0)),
            pl.BlockSpec((1, D), lambda i: (0, 0)),
        ],
        out_specs=[pl.BlockSpec((_RB, D), lambda i: (i, 0)),
                   pl.BlockSpec((_RB, D // 2), lambda i: (i, 0))],
        out_shape=[jax.ShapeDtypeStruct((N, D), jnp.float32),
                   jax.ShapeDtypeStruct((N, D // 2), jnp.uint32)],
    )(h, parts, w, b.reshape(1, D))


def kernel(x, edge_index, edge_attr, batch, We0, be0, W0, b0,
           We1, be1, W1, b1, We2, be2, W2, b2):
    src = edge_index[0].reshape(NW, EPW)
    dst = edge_index[1].reshape(NW, EPW)
    ipad = jnp.zeros((NW, PADE), jnp.int32)
    src = jnp.concatenate([src, ipad], axis=1)
    dst = jnp.concatenate([dst, ipad + N], axis=1)
    pk = jnp.bitwise_or(jnp.left_shift(dst, 16), src).reshape(NW, NSS, KB, C)
    fpad = jnp.zeros((NW, PADE), jnp.float32)
    ea0 = jnp.concatenate([edge_attr[:, 0].reshape(NW, EPW), fpad],
                          axis=1).reshape(NW, NSS, KBC)
    ea1 = jnp.concatenate([edge_attr[:, 1].reshape(NW, EPW), fpad],
                          axis=1).reshape(NW, NSS, KBC)
    zeros = jnp.zeros((N, D), jnp.float32)

    h = x
    hp = _pack_halves(x)
    for We, be, W, b in ((We0, be0, W0, b0), (We1, be1, W1, b1),
                         (We2, be2, W2, b2)):
        parts = _sc_edge(hp, pk, ea0, ea1, We, be, zeros)
        h, hp = _tc_node(h, parts, W, b)
    return h


# speedup vs baseline: 1.0364x; 1.0364x over previous
import functools

import jax
import jax.numpy as jnp
from jax import lax
from jax.experimental import pallas as pl
from jax.experimental.pallas import tpu as pltpu
from jax.experimental.pallas import tpu_sc as plsc

N, E, D = 10000, 320000, 128

NC, NS, L = 2, 16, 16
NW = NC * NS
EPW = E // NW
C = 128
KB = 1
KBC = KB * C
NSS = 80
KCH = NSS * KB
PADE = NSS * KBC - EPW
RPT = 624
RSZ = 640
NSL = D // L
NTRASH = 8


def _sc_edge_body(h_hbm, pk_hbm, ea0_hbm, ea1_hbm, we_hbm, be_hbm,
                  zeros_hbm, out_hbm,
                  pkbuf, srcidx, dstidx, ea0v, ea1v, wev, bev, rows, agg_sh,
                  gsem, esem, ssem, psem):
    cid = lax.axis_index("c")
    sid = lax.axis_index("s")
    wid = sid * NC + cid

    pltpu.sync_copy(zeros_hbm.at[pl.ds(sid * RPT, RSZ)],
                    agg_sh.at[pl.ds(sid * RPT, RSZ)])

    pltpu.sync_copy(we_hbm, wev)
    pltpu.sync_copy(be_hbm, bev)
    w0 = [wev[0, pl.ds(L * s, L)] for s in range(NSL)]
    w1 = [wev[1, pl.ds(L * s, L)] for s in range(NSL)]
    bb = [bev[pl.ds(L * s, L)] for s in range(NSL)]

    plsc.subcore_barrier()

    def fire_pk(bset, ss):
        pltpu.async_copy(pk_hbm.at[wid, ss], pkbuf.at[bset], psem.at[bset])

    def drain_pk(bset):
        pltpu.make_async_copy(pk_hbm.at[0, 0], pkbuf.at[bset],
                              psem.at[bset]).wait()

    def unpack(bset, ss):
        del ss
        for c in range(KB):
            for g in range(C // L):
                v = pkbuf[bset, c, pl.ds(g * L, L)]
                srcidx[bset, c, pl.ds(g * L, L)] = jnp.bitwise_and(v, 0xFFFF)
                dstidx[bset, c, pl.ds(g * L, L)] = jnp.right_shift(v, 16)

    def fire(bset, ss):
        pltpu.async_copy(ea0_hbm.at[wid, ss], ea0v.at[bset], esem.at[bset])
        pltpu.async_copy(ea1_hbm.at[wid, ss], ea1v.at[bset], esem.at[bset])
        for c in range(KB):
            pltpu.async_copy(h_hbm.at[srcidx.at[bset, c]],
                             rows.at[bset, c], gsem.at[bset])

    def drain_in(bset):
        pltpu.make_async_copy(ea0_hbm.at[0, 0], ea0v.at[bset],
                              esem.at[bset]).wait()
        pltpu.make_async_copy(ea1_hbm.at[0, 0], ea1v.at[bset],
                              esem.at[bset]).wait()
        for c in range(KB):
            pltpu.make_async_copy(h_hbm.at[srcidx.at[bset, c]],
                                  rows.at[bset, c], gsem.at[bset]).wait()

    def drain_scatters(bset):
        for c in range(KB):
            pltpu.make_async_copy(rows.at[bset, c],
                                  agg_sh.at[dstidx.at[bset, c]],
                                  ssem.at[bset]).wait()

    def compute(bset, c):
        def group_body(g, carry):
            j0 = g * L
            a0v = ea0v[bset, pl.ds(c * C + j0, L)]
            a1v = ea1v[bset, pl.ds(c * C + j0, L)]
            for jj in range(L):
                a0 = a0v[jj]
                a1 = a1v[jj]
                for s in range(NSL):
                    v = rows[bset, c, j0 + jj, pl.ds(L * s, L)]
                    v = v + (a0 * w0[s] + (a1 * w1[s] + bb[s]))
                    rows[bset, c, j0 + jj, pl.ds(L * s, L)] = jnp.maximum(v, 0.0)
            return carry

        lax.fori_loop(0, C // L, group_body, 0, unroll=False)

    fire_pk(0, 0)
    drain_pk(0)
    unpack(0, 0)
    fire(0, 0)
    fire_pk(1, 1)

    def outer(t, carry):
        for bset in range(2):
            ss = 2 * t + bset
            other = 1 - bset

            @pl.when(ss + 2 < NSS)
            def _():
                fire_pk(bset, ss + 2)

            @pl.when(ss + 1 < NSS)
            def _():
                @pl.when(ss >= 1)
                def _():
                    drain_scatters(other)
                drain_pk(other)
                unpack(other, ss + 1)
                fire(other, ss + 1)

            drain_in(bset)
            for c in range(KB):
                compute(bset, c)
                pltpu.async_copy(rows.at[bset, c],
                                 agg_sh.at[dstidx.at[bset, c]],
                                 ssem.at[bset], add=True)
        return carry

    lax.fori_loop(0, NSS // 2, outer, 0, unroll=False)
    drain_scatters(0)
    drain_scatters(1)

    plsc.subcore_barrier()
    pltpu.sync_copy(agg_sh.at[pl.ds(sid * RPT, RSZ)],
                    out_hbm.at[cid, pl.ds(sid * RPT, RSZ)])


_sc_edge = functools.partial(
    pl.kernel,
    out_type=jax.ShapeDtypeStruct((NC, N, D), jnp.float32),
    mesh=plsc.VectorSubcoreMesh(core_axis_name="c", subcore_axis_name="s",
                                num_cores=NC, num_subcores=NS),
    scratch_types=[
        pltpu.VMEM((2, KB, C), jnp.int32),
        pltpu.VMEM((2, KB, C), jnp.int32),
        pltpu.VMEM((2, KB, C), jnp.int32),
        pltpu.VMEM((2, KBC), jnp.float32),
        pltpu.VMEM((2, KBC), jnp.float32),
        pltpu.VMEM((2, D), jnp.float32),
        pltpu.VMEM((D,), jnp.float32),
        pltpu.VMEM((2, KB, C, D), jnp.float32),
        pltpu.VMEM_SHARED((N + NTRASH, D), jnp.float32),
        pltpu.SemaphoreType.DMA((2,)),
        pltpu.SemaphoreType.DMA((2,)),
        pltpu.SemaphoreType.DMA((2,)),
        pltpu.SemaphoreType.DMA((2,)),
    ],
)(_sc_edge_body)


def _tc_node_body(h_ref, p_ref, w_ref, b_ref, o_ref):
    t = h_ref[...] + p_ref[0] + p_ref[1]
    y = jnp.dot(t, w_ref[...], preferred_element_type=jnp.float32) + b_ref[...]
    o_ref[...] = jnp.where(y >= 0.0, y, 0.01 * y)


_RB = 1000


def _tc_node(h, parts, w, b):
    return pl.pallas_call(
        _tc_node_body,
        grid=(N // _RB,),
        in_specs=[
            pl.BlockSpec((_RB, D), lambda i: (i, 0)),
            pl.BlockSpec((NC, _RB, D), lambda i: (0, i, 0)),
            pl.BlockSpec((D, D), lambda i: (0, 0)),
            pl.BlockSpec((1, D), lambda i: (0, 0)),
        ],
        out_specs=pl.BlockSpec((_RB, D), lambda i: (i, 0)),
        out_shape=jax.ShapeDtypeStruct((N, D), jnp.float32),
    )(h, parts, w, b.reshape(1, D))


def kernel(x, edge_index, edge_attr, batch, We0, be0, W0, b0,
           We1, be1, W1, b1, We2, be2, W2, b2):
    src = edge_index[0].reshape(NW, EPW)
    dst = edge_index[1].reshape(NW, EPW)
    ipad = jnp.zeros((NW, PADE), jnp.int32)
    src = jnp.concatenate([src, ipad], axis=1)
    dst = jnp.concatenate([dst, ipad + N], axis=1)
    pk = jnp.bitwise_or(jnp.left_shift(dst, 16), src).reshape(NW, NSS, KB, C)
    fpad = jnp.zeros((NW, PADE), jnp.float32)
    ea0 = jnp.concatenate([edge_attr[:, 0].reshape(NW, EPW), fpad],
                          axis=1).reshape(NW, NSS, KBC)
    ea1 = jnp.concatenate([edge_attr[:, 1].reshape(NW, EPW), fpad],
                          axis=1).reshape(NW, NSS, KBC)
    zeros = jnp.zeros((N, D), jnp.float32)

    h = x
    for We, be, W, b in ((We0, be0, W0, b0), (We1, be1, W1, b1),
                         (We2, be2, W2, b2)):
        parts = _sc_edge(h, pk, ea0, ea1, We, be, zeros)
        h = _tc_node(h, parts, W, b)
    return h
